# baseline (speedup 1.0000x reference)
import functools
import jax
import jax.numpy as jnp
from jax import lax
from jax.experimental import pallas as pl
from jax.experimental.pallas import tpu as pltpu
from jax.experimental.pallas import tpu_sc as plsc

N = 10000
E = 320000
D = 128
G = 128
C = 10
BN_EPS = 1e-5

NC = 2
NS = 16
LANE = 128

E_ROWS = 2560
E_PAD = E_ROWS * LANE
HEAVY = 160
LIGHT = 0
BLK = 32
PROWS = 5
NP = 10240
ROWS_PER_TILE = NP // NS
GP = 136
BROWS = NP // LANE


def _sc_body(h_hbm, src_hbm, dst_hbm, agg_a, agg_b,
             src_v, dst_v, rows_v, zbuf, sem0, sem1, agg_sh):
  c = lax.axis_index("c")
  s = lax.axis_index("s")

  zero16 = jnp.zeros((16,), jnp.float32)
  for i in range(32):
    for j in range(8):
      zbuf[i, pl.ds(j * 16, 16)] = zero16

  for k in range(ROWS_PER_TILE // 32):
    pltpu.sync_copy(zbuf, agg_sh.at[pl.ds(s * ROWS_PER_TILE + k * 32, 32)])
  plsc.subcore_barrier()

  if True:
    sems = (sem0, sem1)

    def edge_blk(k, carry):
      row0 = pl.multiple_of(
          jnp.where(c == 1, s * HEAVY, NS * HEAVY + s * LIGHT) + k * BLK, BLK)
      pltpu.sync_copy(src_hbm.at[pl.ds(row0, BLK)], src_v)
      pltpu.sync_copy(dst_hbm.at[pl.ds(row0, BLK)], dst_v)
      pltpu.async_copy(h_hbm.at[src_v.at[0]], rows_v.at[0], sem0)
      pltpu.async_copy(h_hbm.at[src_v.at[1]], rows_v.at[1], sem1)
      for i in range(BLK):
        b = i & 1
        pltpu.make_async_copy(h_hbm.at[src_v.at[i]], rows_v.at[b],
                              sems[b]).wait()
        pltpu.sync_copy(rows_v.at[b], agg_sh.at[dst_v.at[i]], add=True)
        if i + 2 < BLK:
          pltpu.async_copy(h_hbm.at[src_v.at[i + 2]], rows_v.at[b], sems[b])
      return carry

    nblk = jnp.where(c == 1, HEAVY // BLK, LIGHT // BLK)
    lax.fori_loop(0, nblk, edge_blk, 0)

  plsc.subcore_barrier()

  @pl.when(c == 0)
  def _():
    pltpu.sync_copy(agg_sh.at[pl.ds(s * ROWS_PER_TILE, ROWS_PER_TILE)],
                    agg_a.at[pl.ds(s * ROWS_PER_TILE, ROWS_PER_TILE)])
  @pl.when(c == 1)
  def _():
    pltpu.sync_copy(agg_sh.at[pl.ds(s * ROWS_PER_TILE, ROWS_PER_TILE)],
                    agg_b.at[pl.ds(s * ROWS_PER_TILE, ROWS_PER_TILE)])


def _make_sc_kernel():
  mesh = plsc.VectorSubcoreMesh(core_axis_name="c", subcore_axis_name="s",
                                num_cores=NC, num_subcores=NS)
  out_type = (
      jax.ShapeDtypeStruct((NP, D), jnp.float32),
      jax.ShapeDtypeStruct((NP, D), jnp.float32),
  )
  scratch = (
      pltpu.VMEM((BLK, LANE), jnp.int32),
      pltpu.VMEM((BLK, LANE), jnp.int32),
      pltpu.VMEM((2, LANE, D), jnp.float32),
      pltpu.VMEM((32, D), jnp.float32),
      pltpu.SemaphoreType.DMA,
      pltpu.SemaphoreType.DMA,
      pltpu.VMEM_SHARED((NP, D), jnp.float32),
  )
  return pl.kernel(_sc_body, out_type=out_type, mesh=mesh,
                   scratch_types=scratch)


_sc_agg = _make_sc_kernel()


def _mlp_body(x_ref, aa_ref, ab_ref, w1_ref, b1_ref, w2_ref, b2_ref,
              bt_ref, o_ref, g_ref):
  h = x_ref[...] + aa_ref[...] + ab_ref[...]
  t = jnp.dot(h, w1_ref[...], preferred_element_type=jnp.float32) + b1_ref[...]
  t = jnp.maximum(t, 0.0)
  t = jnp.dot(t, w2_ref[...], preferred_element_type=jnp.float32) + b2_ref[...]
  hout = jnp.maximum(t, 0.0)
  o_ref[...] = hout
  seg = jax.lax.broadcasted_iota(jnp.int32, (GP, _MLP_BLK), 0)
  mask = (seg == bt_ref[0]).astype(jnp.float32)
  gpart = jnp.dot(mask, hout, preferred_element_type=jnp.float32)
  i = pl.program_id(0)
  @pl.when(i == 0)
  def _():
    g_ref[...] = gpart
  @pl.when(i > 0)
  def _():
    g_ref[...] += gpart


_MLP_BLK = 1024


def _mlp(x, agg_a, agg_b, bt, w1, b1, w2, b2):
  grid = (NP // _MLP_BLK,)
  row_spec = pl.BlockSpec((_MLP_BLK, D), lambda i: (i, 0))
  full = lambda shp: pl.BlockSpec(shp, lambda i: (0, 0))
  return pl.pallas_call(
      _mlp_body,
      grid=grid,
      in_specs=[row_spec, row_spec, row_spec,
                full((D, D)), full((1, D)), full((D, D)), full((1, D)),
                pl.BlockSpec((1, 1, _MLP_BLK), lambda i: (i, 0, 0))],
      out_specs=[row_spec, pl.BlockSpec((GP, D), lambda i: (0, 0))],
      out_shape=[jax.ShapeDtypeStruct((NP, D), jnp.float32),
                 jax.ShapeDtypeStruct((GP, D), jnp.float32)],
  )(x, agg_a, agg_b, w1, b1, w2, b2, bt)


def _head_body(g1_ref, g2_ref, g3_ref, w1_ref, b1_ref, w2_ref, b2_ref, o_ref):
  g1 = g1_ref[0:G, :]
  g2 = g2_ref[0:G, :]
  g3 = g3_ref[0:G, :]
  g = jnp.concatenate((g1, g2, g3), axis=1)
  t = jnp.dot(g, w1_ref[...], preferred_element_type=jnp.float32) + b1_ref[...]
  t = jnp.maximum(t, 0.0)
  logits = jnp.dot(t, w2_ref[...], preferred_element_type=jnp.float32) + b2_ref[...]
  m = jnp.max(logits, axis=1, keepdims=True)
  sh = logits - m
  lse = jnp.log(jnp.sum(jnp.exp(sh), axis=1, keepdims=True))
  o_ref[...] = sh - lse


def _head(g1, g2, g3, w1, b1, w2p, b2p):
  return pl.pallas_call(
      _head_body,
      out_shape=jax.ShapeDtypeStruct((G, D), jnp.float32),
  )(g1, g2, g3, w1, b1, w2p, b2p)


def _fold_bn(p):
  scale = p['bn_g'] * lax.rsqrt(p['bn_rv'] + BN_EPS)
  w1 = p['W1'] * scale[None, :]
  b1 = (p['b1'] - p['bn_rm']) * scale + p['bn_b']
  return w1, b1.reshape(1, D), p['W2'], p['b2'].reshape(1, D)


def kernel(x, edge_index, batch, params):
  src = edge_index[0]
  dst = edge_index[1]
  epad = E_PAD - E
  src_p = jnp.concatenate(
      [src, jnp.zeros((epad,), jnp.int32)]).reshape(E_ROWS, LANE)
  dst_p = jnp.concatenate(
      [dst, jnp.full((epad,), N, jnp.int32)]).reshape(E_ROWS, LANE)
  batch_p = jnp.concatenate(
      [batch, jnp.full((NP - N,), G, jnp.int32)]).reshape(
          NP // _MLP_BLK, 1, _MLP_BLK)
  x_p = jnp.concatenate([x, jnp.zeros((NP - N, D), jnp.float32)], axis=0)

  mlps = [_fold_bn(params[k]) for k in ('c1', 'c2', 'c3')]

  a1, b1x = _sc_agg(x_p, src_p, dst_p)
  h1, g1 = _mlp(x_p, a1, b1x, batch_p, *mlps[0])
  a2, b2x = _sc_agg(h1, src_p, dst_p)
  h2, g2 = _mlp(h1, a2, b2x, batch_p, *mlps[1])
  a3, b3x = _sc_agg(h2, src_p, dst_p)
  h3, g3 = _mlp(h2, a3, b3x, batch_p, *mlps[2])

  w2p = jnp.zeros((3 * D, D), jnp.float32).at[:, :C].set(params['lin2W'])
  b2p = jnp.full((1, D), -1e30, jnp.float32).at[0, :C].set(params['lin2b'])
  out = _head(g1, g2, g3, params['lin1W'], params['lin1b'].reshape(1, 3 * D),
              w2p, b2p)
  return out[:, :C]


# speedup vs baseline: 1.1994x; 1.1994x over previous
import functools
import jax
import jax.numpy as jnp
from jax import lax
from jax.experimental import pallas as pl
from jax.experimental.pallas import tpu as pltpu
from jax.experimental.pallas import tpu_sc as plsc

N = 10000
E = 320000
D = 128
G = 128
C = 10
BN_EPS = 1e-5

NC = 2
NS = 16
LANE = 128

E_ROWS = 2560
E_PAD = E_ROWS * LANE
HEAVY = 112
LIGHT = 48
BLK = 16
PROWS = 5
NP = 10240
ROWS_PER_TILE = NP // NS
GP = 136
BROWS = NP // LANE


def _sc_body(h_hbm, src_hbm, dst_hbm, agg_a, agg_b,
             src_v, dst_v, rows_v, zbuf, sem0, sem1, agg_sh):
  c = lax.axis_index("c")
  s = lax.axis_index("s")

  zero16 = jnp.zeros((16,), jnp.float32)
  for i in range(32):
    for j in range(8):
      zbuf[i, pl.ds(j * 16, 16)] = zero16

  for k in range(ROWS_PER_TILE // 32):
    pltpu.sync_copy(zbuf, agg_sh.at[pl.ds(s * ROWS_PER_TILE + k * 32, 32)])
  plsc.subcore_barrier()

  if True:
    sems = (sem0, sem1)

    def edge_blk(k, carry):
      row0 = pl.multiple_of(
          jnp.where(c == 1, s * HEAVY, NS * HEAVY + s * LIGHT) + k * BLK, BLK)
      pltpu.sync_copy(src_hbm.at[pl.ds(row0, BLK)], src_v)
      pltpu.sync_copy(dst_hbm.at[pl.ds(row0, BLK)], dst_v)
      pltpu.async_copy(h_hbm.at[src_v.at[0]], rows_v.at[0], sem0)
      pltpu.async_copy(h_hbm.at[src_v.at[1]], rows_v.at[1], sem1)
      for i in range(BLK):
        b = i & 1
        pltpu.make_async_copy(h_hbm.at[src_v.at[i]], rows_v.at[b],
                              sems[b]).wait()
        pltpu.sync_copy(rows_v.at[b], agg_sh.at[dst_v.at[i]], add=True)
        if i + 2 < BLK:
          pltpu.async_copy(h_hbm.at[src_v.at[i + 2]], rows_v.at[b], sems[b])
      return carry

    nblk = jnp.where(c == 1, HEAVY // BLK, LIGHT // BLK)
    lax.fori_loop(0, nblk, edge_blk, 0)

  plsc.subcore_barrier()

  @pl.when(c == 0)
  def _():
    pltpu.sync_copy(agg_sh.at[pl.ds(s * ROWS_PER_TILE, ROWS_PER_TILE)],
                    agg_a.at[pl.ds(s * ROWS_PER_TILE, ROWS_PER_TILE)])
  @pl.when(c == 1)
  def _():
    pltpu.sync_copy(agg_sh.at[pl.ds(s * ROWS_PER_TILE, ROWS_PER_TILE)],
                    agg_b.at[pl.ds(s * ROWS_PER_TILE, ROWS_PER_TILE)])


def _make_sc_kernel():
  mesh = plsc.VectorSubcoreMesh(core_axis_name="c", subcore_axis_name="s",
                                num_cores=NC, num_subcores=NS)
  out_type = (
      jax.ShapeDtypeStruct((NP, D), jnp.float32),
      jax.ShapeDtypeStruct((NP, D), jnp.float32),
  )
  scratch = (
      pltpu.VMEM((BLK, LANE), jnp.int32),
      pltpu.VMEM((BLK, LANE), jnp.int32),
      pltpu.VMEM((2, LANE, D), jnp.float32),
      pltpu.VMEM((32, D), jnp.float32),
      pltpu.SemaphoreType.DMA,
      pltpu.SemaphoreType.DMA,
      pltpu.VMEM_SHARED((NP, D), jnp.float32),
  )
  return pl.kernel(_sc_body, out_type=out_type, mesh=mesh,
                   scratch_types=scratch)


_sc_agg = _make_sc_kernel()


def _mlp_body(x_ref, aa_ref, ab_ref, w1_ref, b1_ref, w2_ref, b2_ref,
              bt_ref, o_ref, g_ref):
  h = x_ref[...] + aa_ref[...] + ab_ref[...]
  t = jnp.dot(h, w1_ref[...], preferred_element_type=jnp.float32) + b1_ref[...]
  t = jnp.maximum(t, 0.0)
  t = jnp.dot(t, w2_ref[...], preferred_element_type=jnp.float32) + b2_ref[...]
  hout = jnp.maximum(t, 0.0)
  o_ref[...] = hout
  seg = jax.lax.broadcasted_iota(jnp.int32, (GP, _MLP_BLK), 0)
  mask = (seg == bt_ref[0]).astype(jnp.float32)
  gpart = jnp.dot(mask, hout, preferred_element_type=jnp.float32)
  i = pl.program_id(0)
  @pl.when(i == 0)
  def _():
    g_ref[...] = gpart
  @pl.when(i > 0)
  def _():
    g_ref[...] += gpart


_MLP_BLK = 1024


def _mlp(x, agg_a, agg_b, bt, w1, b1, w2, b2):
  grid = (NP // _MLP_BLK,)
  row_spec = pl.BlockSpec((_MLP_BLK, D), lambda i: (i, 0))
  full = lambda shp: pl.BlockSpec(shp, lambda i: (0, 0))
  return pl.pallas_call(
      _mlp_body,
      grid=grid,
      in_specs=[row_spec, row_spec, row_spec,
                full((D, D)), full((1, D)), full((D, D)), full((1, D)),
                pl.BlockSpec((1, 1, _MLP_BLK), lambda i: (i, 0, 0))],
      out_specs=[row_spec, pl.BlockSpec((GP, D), lambda i: (0, 0))],
      out_shape=[jax.ShapeDtypeStruct((NP, D), jnp.float32),
                 jax.ShapeDtypeStruct((GP, D), jnp.float32)],
  )(x, agg_a, agg_b, w1, b1, w2, b2, bt)


def _head_body(g1_ref, g2_ref, g3_ref, w1_ref, b1_ref, w2_ref, b2_ref, o_ref):
  g1 = g1_ref[0:G, :]
  g2 = g2_ref[0:G, :]
  g3 = g3_ref[0:G, :]
  g = jnp.concatenate((g1, g2, g3), axis=1)
  t = jnp.dot(g, w1_ref[...], preferred_element_type=jnp.float32) + b1_ref[...]
  t = jnp.maximum(t, 0.0)
  logits = jnp.dot(t, w2_ref[...], preferred_element_type=jnp.float32) + b2_ref[...]
  m = jnp.max(logits, axis=1, keepdims=True)
  sh = logits - m
  lse = jnp.log(jnp.sum(jnp.exp(sh), axis=1, keepdims=True))
  o_ref[...] = sh - lse


def _head(g1, g2, g3, w1, b1, w2p, b2p):
  return pl.pallas_call(
      _head_body,
      out_shape=jax.ShapeDtypeStruct((G, D), jnp.float32),
  )(g1, g2, g3, w1, b1, w2p, b2p)


def _fold_bn(p):
  scale = p['bn_g'] * lax.rsqrt(p['bn_rv'] + BN_EPS)
  w1 = p['W1'] * scale[None, :]
  b1 = (p['b1'] - p['bn_rm']) * scale + p['bn_b']
  return w1, b1.reshape(1, D), p['W2'], p['b2'].reshape(1, D)


def kernel(x, edge_index, batch, params):
  src = edge_index[0]
  dst = edge_index[1]
  epad = E_PAD - E
  src_p = jnp.concatenate(
      [src, jnp.zeros((epad,), jnp.int32)]).reshape(E_ROWS, LANE)
  dst_p = jnp.concatenate(
      [dst, jnp.full((epad,), N, jnp.int32)]).reshape(E_ROWS, LANE)
  batch_p = jnp.concatenate(
      [batch, jnp.full((NP - N,), G, jnp.int32)]).reshape(
          NP // _MLP_BLK, 1, _MLP_BLK)
  x_p = jnp.concatenate([x, jnp.zeros((NP - N, D), jnp.float32)], axis=0)

  mlps = [_fold_bn(params[k]) for k in ('c1', 'c2', 'c3')]

  a1, b1x = _sc_agg(x_p, src_p, dst_p)
  h1, g1 = _mlp(x_p, a1, b1x, batch_p, *mlps[0])
  a2, b2x = _sc_agg(h1, src_p, dst_p)
  h2, g2 = _mlp(h1, a2, b2x, batch_p, *mlps[1])
  a3, b3x = _sc_agg(h2, src_p, dst_p)
  h3, g3 = _mlp(h2, a3, b3x, batch_p, *mlps[2])

  w2p = jnp.zeros((3 * D, D), jnp.float32).at[:, :C].set(params['lin2W'])
  b2p = jnp.full((1, D), -1e30, jnp.float32).at[0, :C].set(params['lin2b'])
  out = _head(g1, g2, g3, params['lin1W'], params['lin1b'].reshape(1, 3 * D),
              w2p, b2p)
  return out[:, :C]


# speedup vs baseline: 1.2444x; 1.0375x over previous
import functools
import jax
import jax.numpy as jnp
from jax import lax
from jax.experimental import pallas as pl
from jax.experimental.pallas import tpu as pltpu
from jax.experimental.pallas import tpu_sc as plsc

N = 10000
E = 320000
D = 128
G = 128
C = 10
BN_EPS = 1e-5

NC = 2
NS = 16
LANE = 128

E_ROWS = 2560
E_PAD = E_ROWS * LANE
HEAVY = 128
LIGHT = 32
BLK = 32
NP = 10240
ROWS_PER_TILE = NP // NS
GP = 136
BROWS = NP // LANE


def _sc_body(do_pool, h_hbm, src_hbm, dst_hbm, batch_hbm,
             agg_a, agg_b, g_out,
             src_v, dst_v, bidx_v, rows_v, zbuf, sem0, sem1,
             agg_sh, g_sh):
  c = lax.axis_index("c")
  s = lax.axis_index("s")
  wid = c * NS + s

  zero16 = jnp.zeros((16,), jnp.float32)
  for i in range(8):
    for j in range(8):
      zbuf[i, pl.ds(j * 16, 16)] = zero16

  if True:
    for k in range(ROWS_PER_TILE // 8):
      pltpu.sync_copy(zbuf, agg_sh.at[pl.ds(s * ROWS_PER_TILE + k * 8, 8)])
    if do_pool:
      @pl.when(s == 0)
      def _():
        for k in range(GP // 8):
          pltpu.sync_copy(zbuf, g_sh.at[pl.ds(k * 8, 8)])
    plsc.subcore_barrier()

    sems = (sem0, sem1)
    base_row = jnp.where(c == 0, s * HEAVY, NS * HEAVY + s * LIGHT)
    nblk = jnp.where(c == 0, HEAVY // BLK, LIGHT // BLK)

    def edge_blk(k, carry):
      row0 = pl.multiple_of(base_row + k * BLK, BLK)
      pltpu.sync_copy(src_hbm.at[pl.ds(row0, BLK)], src_v)
      pltpu.sync_copy(dst_hbm.at[pl.ds(row0, BLK)], dst_v)
      pltpu.async_copy(h_hbm.at[src_v.at[0]], rows_v.at[0], sem0)
      pltpu.async_copy(h_hbm.at[src_v.at[1]], rows_v.at[1], sem1)
      for i in range(BLK):
        b = i & 1
        pltpu.make_async_copy(h_hbm.at[src_v.at[i]], rows_v.at[b],
                              sems[b]).wait()
        pltpu.sync_copy(rows_v.at[b], agg_sh.at[dst_v.at[i]], add=True)
        if i + 2 < BLK:
          pltpu.async_copy(h_hbm.at[src_v.at[i + 2]], rows_v.at[b], sems[b])
      return carry

    lax.fori_loop(0, nblk, edge_blk, 0)

    if do_pool:
      pltpu.sync_copy(batch_hbm.at[wid], bidx_v)
      for j in range(3):
        nb_row = jnp.where(wid < 16, 3 * wid + j,
                           jnp.where(j < 2, 48 + 2 * (wid - 16) + j, 0))
        pltpu.async_copy(h_hbm.at[pl.ds(nb_row * LANE, LANE)], rows_v.at[0],
                         sem0).wait()
        pltpu.sync_copy(rows_v.at[0], g_sh.at[bidx_v.at[j]], add=True)

    plsc.subcore_barrier()

    @pl.when(c == 0)
    def _():
      pltpu.sync_copy(agg_sh.at[pl.ds(s * ROWS_PER_TILE, ROWS_PER_TILE)],
                      agg_a.at[pl.ds(s * ROWS_PER_TILE, ROWS_PER_TILE)])
    @pl.when(c == 1)
    def _():
      pltpu.sync_copy(agg_sh.at[pl.ds(s * ROWS_PER_TILE, ROWS_PER_TILE)],
                      agg_b.at[pl.ds(s * ROWS_PER_TILE, ROWS_PER_TILE)])
    if do_pool:
      @pl.when(s == 0)
      def _():
        pltpu.sync_copy(g_sh, g_out.at[c])


def _make_sc_kernel(do_pool):
  mesh = plsc.VectorSubcoreMesh(core_axis_name="c", subcore_axis_name="s",
                                num_cores=NC, num_subcores=NS)
  out_type = [
      jax.ShapeDtypeStruct((NP, D), jnp.float32),
      jax.ShapeDtypeStruct((NP, D), jnp.float32),
  ]
  if do_pool:
    out_type.append(jax.ShapeDtypeStruct((NC, GP, D), jnp.float32))
  scratch = [
      pltpu.VMEM((BLK, LANE), jnp.int32),
      pltpu.VMEM((BLK, LANE), jnp.int32),
      pltpu.VMEM((3, LANE), jnp.int32),
      pltpu.VMEM((2, LANE, D), jnp.float32),
      pltpu.VMEM((8, D), jnp.float32),
      pltpu.SemaphoreType.DMA,
      pltpu.SemaphoreType.DMA,
      pltpu.VMEM_SHARED((NP, D), jnp.float32),
      pltpu.VMEM_SHARED((GP, D), jnp.float32),
  ]

  if do_pool:
    def body(h, src, dst, batch, agg_a, agg_b, g_out, *scr):
      _sc_body(True, h, src, dst, batch, agg_a, agg_b, g_out, *scr)
  else:
    def body(h, src, dst, batch, agg_a, agg_b, *scr):
      _sc_body(False, h, src, dst, batch, agg_a, agg_b, None, *scr)

  return pl.kernel(body, out_type=tuple(out_type), mesh=mesh,
                   scratch_types=tuple(scratch))


_sc_agg = _make_sc_kernel(False)
_sc_agg_pool = _make_sc_kernel(True)


def _pool_only_body(h_hbm, batch_hbm, g_out, bidx_v, hrow_v, zbuf, sem, g_sh):
  c = lax.axis_index("c")
  s = lax.axis_index("s")
  wid = c * NS + s
  zero16 = jnp.zeros((16,), jnp.float32)
  for i in range(8):
    for j in range(8):
      zbuf[i, pl.ds(j * 16, 16)] = zero16

  if True:
    @pl.when(s == 0)
    def _():
      for k in range(GP // 8):
        pltpu.sync_copy(zbuf, g_sh.at[pl.ds(k * 8, 8)])
    plsc.subcore_barrier()

    pltpu.sync_copy(batch_hbm.at[wid], bidx_v)
    for j in range(3):
      nb_row = jnp.where(wid < 16, 3 * wid + j,
                         jnp.where(j < 2, 48 + 2 * (wid - 16) + j, 0))
      pltpu.async_copy(h_hbm.at[pl.ds(nb_row * LANE, LANE)], hrow_v,
                       sem).wait()
      pltpu.sync_copy(hrow_v, g_sh.at[bidx_v.at[j]], add=True)
    plsc.subcore_barrier()

    @pl.when(s == 0)
    def _():
      pltpu.sync_copy(g_sh, g_out.at[c])


_sc_pool = pl.kernel(
    _pool_only_body,
    out_type=jax.ShapeDtypeStruct((NC, GP, D), jnp.float32),
    mesh=plsc.VectorSubcoreMesh(core_axis_name="c", subcore_axis_name="s",
                                num_cores=NC, num_subcores=NS),
    scratch_types=(
        pltpu.VMEM((3, LANE), jnp.int32),
        pltpu.VMEM((LANE, D), jnp.float32),
        pltpu.VMEM((8, D), jnp.float32),
        pltpu.SemaphoreType.DMA,
        pltpu.VMEM_SHARED((GP, D), jnp.float32),
    ),
)


def _mlp_body(x_ref, aa_ref, ab_ref, w1_ref, b1_ref, w2_ref, b2_ref, o_ref):
  h = x_ref[...] + aa_ref[...] + ab_ref[...]
  t = jnp.dot(h, w1_ref[...], preferred_element_type=jnp.float32) + b1_ref[...]
  t = jnp.maximum(t, 0.0)
  t = jnp.dot(t, w2_ref[...], preferred_element_type=jnp.float32) + b2_ref[...]
  o_ref[...] = jnp.maximum(t, 0.0)


_MLP_BLK = 1024


def _mlp(x, agg_a, agg_b, w1, b1, w2, b2):
  grid = (NP // _MLP_BLK,)
  row_spec = pl.BlockSpec((_MLP_BLK, D), lambda i: (i, 0))
  full = lambda shp: pl.BlockSpec(shp, lambda i: (0, 0))
  return pl.pallas_call(
      _mlp_body,
      grid=grid,
      in_specs=[row_spec, row_spec, row_spec,
                full((D, D)), full((1, D)), full((D, D)), full((1, D))],
      out_specs=row_spec,
      out_shape=jax.ShapeDtypeStruct((NP, D), jnp.float32),
  )(x, agg_a, agg_b, w1, b1, w2, b2)


def _head_body(g1_ref, g2_ref, g3_ref, w1_ref, b1_ref, w2_ref, b2_ref, o_ref):
  g1 = g1_ref[0, 0:G, :] + g1_ref[1, 0:G, :]
  g2 = g2_ref[0, 0:G, :] + g2_ref[1, 0:G, :]
  g3 = g3_ref[0, 0:G, :] + g3_ref[1, 0:G, :]
  g = jnp.concatenate((g1, g2, g3), axis=1)
  t = jnp.dot(g, w1_ref[...], preferred_element_type=jnp.float32) + b1_ref[...]
  t = jnp.maximum(t, 0.0)
  logits = jnp.dot(t, w2_ref[...], preferred_element_type=jnp.float32) + b2_ref[...]
  m = jnp.max(logits, axis=1, keepdims=True)
  sh = logits - m
  lse = jnp.log(jnp.sum(jnp.exp(sh), axis=1, keepdims=True))
  o_ref[...] = sh - lse


def _head(g1, g2, g3, w1, b1, w2p, b2p):
  return pl.pallas_call(
      _head_body,
      out_shape=jax.ShapeDtypeStruct((G, D), jnp.float32),
  )(g1, g2, g3, w1, b1, w2p, b2p)


def _fold_bn(p):
  scale = p['bn_g'] * lax.rsqrt(p['bn_rv'] + BN_EPS)
  w1 = p['W1'] * scale[None, :]
  b1 = (p['b1'] - p['bn_rm']) * scale + p['bn_b']
  return w1, b1.reshape(1, D), p['W2'], p['b2'].reshape(1, D)


def kernel(x, edge_index, batch, params):
  src = edge_index[0]
  dst = edge_index[1]
  epad = E_PAD - E
  src_p = jnp.concatenate(
      [src, jnp.zeros((epad,), jnp.int32)]).reshape(E_ROWS, LANE)
  dst_p = jnp.concatenate(
      [dst, jnp.full((epad,), N, jnp.int32)]).reshape(E_ROWS, LANE)
  bp = jnp.concatenate(
      [batch, jnp.full((NP - N,), G, jnp.int32)]).reshape(BROWS, LANE)
  batch_p = jnp.concatenate([
      bp[:48].reshape(16, 3, LANE),
      jnp.concatenate([bp[48:].reshape(16, 2, LANE),
                       jnp.full((16, 1, LANE), G, jnp.int32)], axis=1),
  ], axis=0)
  x_p = jnp.concatenate([x, jnp.zeros((NP - N, D), jnp.float32)], axis=0)

  mlps = [_fold_bn(params[k]) for k in ('c1', 'c2', 'c3')]

  a1, b1_ = _sc_agg(x_p, src_p, dst_p, batch_p)
  h1 = _mlp(x_p, a1, b1_, *mlps[0])
  a2, b2_, g1 = _sc_agg_pool(h1, src_p, dst_p, batch_p)
  h2 = _mlp(h1, a2, b2_, *mlps[1])
  a3, b3_, g2 = _sc_agg_pool(h2, src_p, dst_p, batch_p)
  h3 = _mlp(h2, a3, b3_, *mlps[2])
  g3 = _sc_pool(h3, batch_p)

  w2p = jnp.zeros((3 * D, D), jnp.float32).at[:, :C].set(params['lin2W'])
  b2p = jnp.full((1, D), -1e30, jnp.float32).at[0, :C].set(params['lin2b'])
  out = _head(g1, g2, g3, params['lin1W'], params['lin1b'].reshape(1, 3 * D),
              w2p, b2p)
  return out[:, :C]
